# Initial kernel scaffold; baseline (speedup 1.0000x reference)
import jax
import jax.numpy as jnp
from jax.experimental import pallas as pl


def kernel(xd, xt, edge_index, batch, y, emb, conv_w, conv_b, fc1_xd_w, fc1_xd_b, gin0_w1, gin0_b1, gin0_w2, gin0_b2, bn0_g, bn0_b, gin1_w1, gin1_b1, gin1_w2, gin1_b2, bn1_g, bn1_b, gin2_w1, gin2_b1, gin2_w2, gin2_b2, bn2_g, bn2_b, gin3_w1, gin3_b1, gin3_w2, gin3_b2, bn3_g, bn3_b, gin4_w1, gin4_b1, gin4_w2, gin4_b2, bn4_g, bn4_b, fc1_xt_w, fc1_xt_b, cls_w1, cls_b1, cls_w2, cls_b2, cls_w3, cls_b3):
    raise NotImplementedError("write your pallas kernel here")


# speedup vs baseline: 7.9621x; 7.9621x over previous
import functools

import jax
import jax.numpy as jnp
from jax import lax
from jax.experimental import pallas as pl
from jax.experimental.pallas import tpu as pltpu
from jax.experimental.pallas import tpu_sc as plsc

N = 50000
E = 1600000
B = 1024
NC, NS = 2, 16
NW = NC * NS

R_G = 392
IB_G = 8
NP_G = 50048
ZR_G = 184

R_D = 32
IB_D = 8
NP_D = 1152
ZR_D = 72


def _make_sc_segsum(n_table, feat, rows_pw, ib, np_out, zr):
    stripe = np_out // NS
    assert rows_pw % ib == 0 and stripe % zr == 0
    mesh = plsc.VectorSubcoreMesh(core_axis_name="c", subcore_axis_name="s")

    @functools.partial(
        pl.kernel,
        out_type=[jax.ShapeDtypeStruct((np_out, feat), jnp.float32),
                  jax.ShapeDtypeStruct((np_out, feat), jnp.float32)],
        mesh=mesh,
        compiler_params=pltpu.CompilerParams(use_tc_tiling_on_sc=False),
        scratch_types=[
            pltpu.VMEM((ib, 128), jnp.int32),
            pltpu.VMEM((ib, 128), jnp.int32),
            pltpu.VMEM((128, feat), jnp.float32),
            pltpu.VMEM((zr, feat), jnp.float32),
            pltpu.VMEM_SHARED((np_out, feat), jnp.float32),
        ],
    )
    def k(table_hbm, src_hbm, dst_hbm, zeros_hbm, out0_hbm, out1_hbm,
          sbuf, dbuf, rows, zbuf, acc):
        c = lax.axis_index("c")
        s = lax.axis_index("s")
        wid = c * NS + s
        pltpu.sync_copy(zeros_hbm, zbuf)

        @pl.loop(0, stripe // zr)
        def _(t):
            pltpu.sync_copy(zbuf, acc.at[pl.ds(s * stripe + t * zr, zr)])

        plsc.subcore_barrier()

        @pl.loop(0, rows_pw // ib)
        def _(o):
            row0 = wid * rows_pw + o * ib
            pltpu.sync_copy(src_hbm.at[pl.ds(row0, ib)], sbuf)
            pltpu.sync_copy(dst_hbm.at[pl.ds(row0, ib)], dbuf)
            for j in range(ib):
                pltpu.sync_copy(table_hbm.at[sbuf.at[j]], rows)
                pltpu.sync_copy(rows, acc.at[dbuf.at[j]], add=True)

        plsc.subcore_barrier()

        @pl.when(c == 0)
        def _():
            pltpu.sync_copy(acc.at[pl.ds(s * stripe, stripe)],
                            out0_hbm.at[pl.ds(s * stripe, stripe)])

        @pl.when(c == 1)
        def _():
            pltpu.sync_copy(acc.at[pl.ds(s * stripe, stripe)],
                            out1_hbm.at[pl.ds(s * stripe, stripe)])

    return k


_sc_gin = _make_sc_segsum(N, 32, R_G, IB_G, NP_G, ZR_G)
_sc_drug = _make_sc_segsum(6500, 128, R_D, IB_D, NP_D, ZR_D)

_RM = 2000


def _mm_nodes(x, wt):
    fin = x.shape[1]

    def body(x_ref, w_ref, o_ref):
        o_ref[...] = jnp.dot(x_ref[...], w_ref[...],
                             preferred_element_type=jnp.float32)

    return pl.pallas_call(
        body,
        grid=(N // _RM,),
        in_specs=[pl.BlockSpec((_RM, fin), lambda i: (i, 0)),
                  pl.BlockSpec((fin, 32), lambda i: (0, 0))],
        out_specs=pl.BlockSpec((_RM, 32), lambda i: (i, 0)),
        out_shape=jax.ShapeDtypeStruct((N, 32), jnp.float32),
    )(x, wt)


def _gin_mlp(p, q0, q1, b1, w2t, b2):
    def body(p_ref, q0_ref, q1_ref, b1_ref, w2t_ref, b2_ref,
             r_ref, s1_ref, s2_ref):
        i = pl.program_id(0)
        z = jnp.maximum(p_ref[...] + q0_ref[...] + q1_ref[...] + b1_ref[...],
                        0.0)
        r = jnp.maximum(jnp.dot(z, w2t_ref[...],
                                preferred_element_type=jnp.float32)
                        + b2_ref[...], 0.0)
        r_ref[...] = r

        @pl.when(i == 0)
        def _():
            s1_ref[...] = jnp.zeros_like(s1_ref)
            s2_ref[...] = jnp.zeros_like(s2_ref)

        s1_ref[...] += jnp.sum(r, axis=0, keepdims=True)
        s2_ref[...] += jnp.sum(r * r, axis=0, keepdims=True)

    return pl.pallas_call(
        body,
        grid=(N // _RM,),
        in_specs=[pl.BlockSpec((_RM, 32), lambda i: (i, 0)),
                  pl.BlockSpec((_RM, 32), lambda i: (i, 0)),
                  pl.BlockSpec((_RM, 32), lambda i: (i, 0)),
                  pl.BlockSpec((1, 32), lambda i: (0, 0)),
                  pl.BlockSpec((32, 32), lambda i: (0, 0)),
                  pl.BlockSpec((1, 32), lambda i: (0, 0))],
        out_specs=[pl.BlockSpec((_RM, 32), lambda i: (i, 0)),
                   pl.BlockSpec((1, 32), lambda i: (0, 0)),
                   pl.BlockSpec((1, 32), lambda i: (0, 0))],
        out_shape=[jax.ShapeDtypeStruct((N, 32), jnp.float32),
                   jax.ShapeDtypeStruct((1, 32), jnp.float32),
                   jax.ShapeDtypeStruct((1, 32), jnp.float32)],
    )(p, q0, q1, b1, w2t, b2)


def _bn_next(r, s1, s2, g, bb, w1t):
    def body(r_ref, s1_ref, s2_ref, g_ref, bb_ref, w1t_ref, o_ref):
        m = s1_ref[...] * (1.0 / N)
        v = s2_ref[...] * (1.0 / N) - m * m
        h = (r_ref[...] - m) * lax.rsqrt(v + 1e-5) * g_ref[...] + bb_ref[...]
        o_ref[...] = jnp.dot(h, w1t_ref[...],
                             preferred_element_type=jnp.float32)

    return pl.pallas_call(
        body,
        grid=(N // _RM,),
        in_specs=[pl.BlockSpec((_RM, 32), lambda i: (i, 0)),
                  pl.BlockSpec((1, 32), lambda i: (0, 0)),
                  pl.BlockSpec((1, 32), lambda i: (0, 0)),
                  pl.BlockSpec((1, 32), lambda i: (0, 0)),
                  pl.BlockSpec((1, 32), lambda i: (0, 0)),
                  pl.BlockSpec((32, 32), lambda i: (0, 0))],
        out_specs=pl.BlockSpec((_RM, 32), lambda i: (i, 0)),
        out_shape=jax.ShapeDtypeStruct((N, 32), jnp.float32),
    )(r, s1, s2, g, bb, w1t)


_RP = 1000


def _pool(h, batch3):
    def body(h_ref, b_ref, ps_ref, pc_ref):
        i = pl.program_id(0)

        @pl.when(i == 0)
        def _():
            ps_ref[...] = jnp.zeros_like(ps_ref)
            pc_ref[...] = jnp.zeros_like(pc_ref)

        ids = b_ref[0]
        ohT = (lax.broadcasted_iota(jnp.int32, (B, _RP), 0)
               == ids).astype(jnp.float32)
        ps_ref[...] += jnp.dot(ohT, h_ref[...],
                               preferred_element_type=jnp.float32)
        pc_ref[...] += jnp.sum(ohT, axis=1, keepdims=True)

    return pl.pallas_call(
        body,
        grid=(N // _RP,),
        in_specs=[pl.BlockSpec((_RP, 32), lambda i: (i, 0)),
                  pl.BlockSpec((1, 1, _RP), lambda i: (i, 0, 0))],
        out_specs=[pl.BlockSpec((B, 32), lambda i: (0, 0)),
                   pl.BlockSpec((B, 1), lambda i: (0, 0))],
        out_shape=[jax.ShapeDtypeStruct((B, 32), jnp.float32),
                   jax.ShapeDtypeStruct((B, 1), jnp.float32)],
    )(h, batch3)


def _head(ps, pc, d0, d1, beff, fxt_t, fxt_b, w1a, w1b, cb1, w2t, cb2,
          w3t, cb3):
    def body(ps_ref, pc_ref, d0_ref, d1_ref, beff_ref, fxt_t_ref, fxt_b_ref,
             w1a_ref, w1b_ref, cb1_ref, w2t_ref, cb2_ref, w3t_ref, cb3_ref,
             o_ref):
        hp = ps_ref[...] / jnp.maximum(pc_ref[...], 1.0)
        ht = jnp.maximum(jnp.dot(hp, fxt_t_ref[...],
                                 preferred_element_type=jnp.float32)
                         + fxt_b_ref[...], 0.0)
        xdo = d0_ref[...] + d1_ref[...] + beff_ref[...]
        z1 = jnp.maximum(jnp.dot(xdo, w1a_ref[...],
                                 preferred_element_type=jnp.float32)
                         + jnp.dot(ht, w1b_ref[...],
                                   preferred_element_type=jnp.float32)
                         + cb1_ref[...], 0.0)
        z2 = jnp.maximum(jnp.dot(z1, w2t_ref[...],
                                 preferred_element_type=jnp.float32)
                         + cb2_ref[...], 0.0)
        o_ref[...] = jnp.dot(z2, w3t_ref[...],
                             preferred_element_type=jnp.float32) + cb3_ref[...]

    return pl.pallas_call(
        body,
        grid=(1,),
        in_specs=[pl.BlockSpec((B, 32), lambda i: (0, 0)),
                  pl.BlockSpec((B, 1), lambda i: (0, 0)),
                  pl.BlockSpec((B, 128), lambda i: (0, 0)),
                  pl.BlockSpec((B, 128), lambda i: (0, 0)),
                  pl.BlockSpec((1, 128), lambda i: (0, 0)),
                  pl.BlockSpec((32, 128), lambda i: (0, 0)),
                  pl.BlockSpec((1, 128), lambda i: (0, 0)),
                  pl.BlockSpec((128, B), lambda i: (0, 0)),
                  pl.BlockSpec((128, B), lambda i: (0, 0)),
                  pl.BlockSpec((1, B), lambda i: (0, 0)),
                  pl.BlockSpec((B, 256), lambda i: (0, 0)),
                  pl.BlockSpec((1, 256), lambda i: (0, 0)),
                  pl.BlockSpec((256, 1), lambda i: (0, 0)),
                  pl.BlockSpec((1, 1), lambda i: (0, 0))],
        out_specs=pl.BlockSpec((B, 1), lambda i: (0, 0)),
        out_shape=jax.ShapeDtypeStruct((B, 1), jnp.float32),
    )(ps, pc, d0, d1, beff, fxt_t, fxt_b, w1a, w1b, cb1, w2t, cb2,
      w3t, cb3)


def kernel(xd, xt, edge_index, batch, y, emb, conv_w, conv_b, fc1_xd_w,
           fc1_xd_b, gin0_w1, gin0_b1, gin0_w2, gin0_b2, bn0_g, bn0_b,
           gin1_w1, gin1_b1, gin1_w2, gin1_b2, bn1_g, bn1_b, gin2_w1,
           gin2_b1, gin2_w2, gin2_b2, bn2_g, bn2_b, gin3_w1, gin3_b1,
           gin3_w2, gin3_b2, bn3_g, bn3_b, gin4_w1, gin4_b1, gin4_w2,
           gin4_b2, bn4_g, bn4_b, fc1_xt_w, fc1_xt_b, cls_w1, cls_b1,
           cls_w2, cls_b2, cls_w3, cls_b3):
    f32 = jnp.float32
    i32 = jnp.int32

    fc3 = fc1_xd_w.reshape(128, 32, 121)
    embw = jnp.stack([emb[:, k:k + 121] for k in range(8)], axis=1)
    tbl = jnp.einsum('vkp,fop->vkof', embw, fc3)
    tbl = jnp.einsum('oik,vkof->ivf', conv_w, tbl).reshape(100 * 65, 128)
    beff = (fc1_xd_b
            + jnp.einsum('o,fo->f', conv_b, fc3.sum(-1))).reshape(1, 128)
    dpad = NW * R_D * 128 - B * 100
    didx = jnp.concatenate(
        [(jnp.arange(100, dtype=i32)[None, :] * 65
          + xd.astype(i32)).reshape(-1), jnp.zeros((dpad,), i32)]
    ).reshape(NW * R_D, 128)
    ddst = jnp.concatenate(
        [jnp.arange(B * 100, dtype=i32) // 100, jnp.full((dpad,), B, i32)]
    ).reshape(NW * R_D, 128)
    zd = jnp.zeros((ZR_D, 128), f32)
    d0, d1 = _sc_drug(tbl, didx, ddst, zd)

    pad = NW * R_G * 128 - E
    src = jnp.concatenate(
        [edge_index[0].astype(i32), jnp.zeros((pad,), i32)]
    ).reshape(NW * R_G, 128)
    dst = jnp.concatenate(
        [edge_index[1].astype(i32), jnp.full((pad,), N, i32)]
    ).reshape(NW * R_G, 128)
    zg = jnp.zeros((ZR_G, 32), f32)

    gparams = [(gin0_w1, gin0_b1, gin0_w2, gin0_b2, bn0_g, bn0_b),
               (gin1_w1, gin1_b1, gin1_w2, gin1_b2, bn1_g, bn1_b),
               (gin2_w1, gin2_b1, gin2_w2, gin2_b2, bn2_g, bn2_b),
               (gin3_w1, gin3_b1, gin3_w2, gin3_b2, bn3_g, bn3_b),
               (gin4_w1, gin4_b1, gin4_w2, gin4_b2, bn4_g, bn4_b)]

    p = _mm_nodes(xt, gparams[0][0].T)
    for li, (w1, b1, w2, b2, g, bb) in enumerate(gparams):
        q0, q1 = _sc_gin(p, src, dst, zg)
        r, s1, s2 = _gin_mlp(p, q0, q1, b1.reshape(1, 32), w2.T,
                             b2.reshape(1, 32))
        w1t_next = gparams[li + 1][0].T if li < 4 else jnp.eye(32, dtype=f32)
        p = _bn_next(r, s1, s2, g.reshape(1, 32), bb.reshape(1, 32), w1t_next)

    batch3 = batch.astype(i32).reshape(N // _RP, 1, _RP)
    ps, pc = _pool(p, batch3)
    out2d = _head(ps, pc, d0, d1, beff, fc1_xt_w.T,
                  fc1_xt_b.reshape(1, 128), cls_w1[:, :128].T,
                  cls_w1[:, 128:].T, cls_b1.reshape(1, B), cls_w2.T,
                  cls_b2.reshape(1, 256), cls_w3.T, cls_b3.reshape(1, 1))
    return (out2d[:, 0], y)
